# baseline (speedup 1.0000x reference)
import functools

import jax
import jax.numpy as jnp
from jax import lax
from jax.experimental import pallas as pl

K = 20
EPS = 1e-5


def _relu_pallas(x):
    def body(x_ref, o_ref):
        o_ref[...] = jnp.maximum(x_ref[...], 0.0)

    return pl.pallas_call(
        body,
        out_shape=jax.ShapeDtypeStruct(x.shape, x.dtype),
    )(x)


def _edge_layer(h, W, b, g, e):
    B, D, N = h.shape
    Wc = W[:, :D]
    Wn = W[:, D:]
    r = jnp.sum(h * h, axis=1)
    s = jnp.einsum('bdn,bdm->bnm', h, h)
    t = s - 0.5 * r[:, None, :]
    _, idx = lax.top_k(t, K)

    a = jnp.einsum('oc,bcn->bon', Wc - Wn, h) + b[None, :, None]
    bb = jnp.einsum('oc,bcn->bon', Wn, h)

    gath = jax.vmap(lambda f, i: f[:, i])(bb, idx)
    S1 = jnp.sum(gath, axis=-1)
    S2 = jnp.sum(gath * gath, axis=-1)
    Mx = jnp.max(gath, axis=-1)
    Mn = jnp.min(gath, axis=-1)

    cnt = B * N * K
    mean = (K * jnp.sum(a, axis=(0, 2)) + jnp.sum(S1, axis=(0, 2))) / cnt
    Ey2 = (K * jnp.sum(a * a, axis=(0, 2))
           + 2.0 * jnp.sum(a * S1, axis=(0, 2))
           + jnp.sum(S2, axis=(0, 2))) / cnt
    var = Ey2 - mean * mean
    scale = g / jnp.sqrt(var + EPS)
    shift = e - mean * scale
    Msel = jnp.where(scale[None, :, None] >= 0, Mx, Mn)
    y = scale[None, :, None] * (a + Msel) + shift[None, :, None]
    return _relu_pallas(y)


def kernel(x, W0, b0, g0, e0, W1, b1, g1, e1, W2, b2, g2, e2, Wf, bf, gf, ef):
    h = x
    feats = []
    for (W, b, g, e) in ((W0, b0, g0, e0), (W1, b1, g1, e1), (W2, b2, g2, e2)):
        h = _edge_layer(h, W, b, g, e)
        feats.append(h)
    cat = jnp.concatenate(feats, axis=1)
    y = jnp.einsum('oc,bcn->bon', Wf, cat) + bf[None, :, None]
    mean = jnp.mean(y, axis=(0, 2))
    var = jnp.var(y, axis=(0, 2))
    y = (y - mean[None, :, None]) / jnp.sqrt(var[None, :, None] + EPS)
    y = y * gf[None, :, None] + ef[None, :, None]
    return _relu_pallas(y)


# speedup vs baseline: 6.4905x; 6.4905x over previous
import functools

import jax
import jax.numpy as jnp
from jax import lax
from jax.experimental import pallas as pl
from jax.experimental.pallas import tpu as pltpu
from jax.experimental.pallas import tpu_sc as plsc

KNN = 20
EPS = 1e-5
B, N = 16, 2048
C = 64

_NC, _NS = 2, 16
_NW = _NC * _NS


def _topk_body(hr_ref, hf_ref, idx_ref, *, M, Dp):
    hr = hr_ref[0]
    hf = hf_ref[0]
    b = pl.program_id(0)
    sq_r = jnp.sum(hr * hr, axis=1, keepdims=True)
    sq_f = jnp.sum(hf * hf, axis=1)[None, :]
    s = lax.dot_general(hr.astype(jnp.bfloat16), hf.astype(jnp.bfloat16),
                        (((1,), (1,)), ((), ())),
                        preferred_element_type=jnp.float32)
    dist = (sq_r + (-2.0 * s)) + sq_f
    iota = lax.broadcasted_iota(jnp.int32, (M, N), 1)
    kiota = lax.broadcasted_iota(jnp.int32, (M, 32), 1)
    base = b * N

    def step(t, carry):
        work, acc = carry
        m = jnp.min(work, axis=1, keepdims=True)
        cand = jnp.where(work == m, iota, N)
        j = jnp.min(cand, axis=1, keepdims=True)
        acc = jnp.where(kiota == t, j + base, acc)
        work = jnp.where(iota == j, jnp.float32(jnp.inf), work)
        return work, acc

    _, acc = lax.fori_loop(0, KNN, step, (dist, jnp.zeros((M, 32), jnp.int32)))
    idx_ref[0] = acc[:, :KNN]


def _topk_call(h, Dp, M=128):
    grid = (B, N // M)
    return pl.pallas_call(
        functools.partial(_topk_body, M=M, Dp=Dp),
        grid=grid,
        in_specs=[
            pl.BlockSpec((1, M, Dp), lambda b, t: (b, t, 0)),
            pl.BlockSpec((1, N, Dp), lambda b, t: (b, 0, 0)),
        ],
        out_specs=pl.BlockSpec((1, M, KNN), lambda b, t: (b, t, 0)),
        out_shape=jax.ShapeDtypeStruct((B, N, KNN), jnp.int32),
    )(h, h)


def _sc_gather(table, idxf, Dp):
    P = 64
    CHUNK = P * KNN
    per_w = (B * N) // _NW
    n_chunks = per_w // P
    SUB = 128
    n_sub = CHUNK // SUB

    mesh = plsc.VectorSubcoreMesh(core_axis_name="c", subcore_axis_name="s")

    @functools.partial(
        pl.kernel,
        out_type=jax.ShapeDtypeStruct((B * N * KNN, Dp), jnp.float32),
        mesh=mesh,
        compiler_params=pltpu.CompilerParams(use_tc_tiling_on_sc=False),
        scratch_types=[
            pltpu.VMEM((CHUNK,), jnp.int32),
            pltpu.VMEM((CHUNK, Dp), jnp.float32),
            pltpu.SemaphoreType.DMA,
        ],
    )
    def gk(table_hbm, idx_hbm, out_hbm, idx_v, rows_v, sem):
        wid = lax.axis_index("s") * _NC + lax.axis_index("c")
        w_off = wid * per_w * KNN

        def chunk_body(ci, _):
            off = w_off + ci * CHUNK
            pltpu.sync_copy(idx_hbm.at[pl.ds(off, CHUNK)], idx_v)
            descs = []
            for j in range(n_sub):
                descs.append(pltpu.async_copy(
                    table_hbm.at[idx_v.at[pl.ds(j * SUB, SUB)]],
                    rows_v.at[pl.ds(j * SUB, SUB)],
                    sem))
            for d in descs:
                d.wait()
            pltpu.sync_copy(rows_v, out_hbm.at[pl.ds(off, CHUNK)])
            return 0

        lax.fori_loop(0, n_chunks, chunk_body, 0)

    return gk(table, idxf)


def _conv_body(g_ref, h_ref, wct_ref, wnt_ref, bias_ref, mx_ref, mn_ref, ps_ref,
               *, M, Dp):
    hr = h_ref[0]
    cen = lax.dot_general(hr.astype(jnp.bfloat16), wct_ref[...],
                          (((1,), (0,)), ((), ())),
                          preferred_element_type=jnp.float32)
    cen = cen + bias_ref[...][0][None, :]
    g3 = g_ref[0].reshape(M, KNN, Dp)
    diff = (g3 - hr[:, None, :]).astype(jnp.bfloat16).reshape(M * KNN, Dp)
    yn = lax.dot_general(diff, wnt_ref[...],
                         (((1,), (0,)), ((), ())),
                         preferred_element_type=jnp.float32)
    y = yn.reshape(M, KNN, C) + cen[:, None, :]
    s1 = jnp.sum(y, axis=1)
    s2 = jnp.sum(y * y, axis=1)
    mx_ref[0] = jnp.max(y, axis=1)
    mn_ref[0] = jnp.min(y, axis=1)
    ps_ref[0, 0] = jnp.concatenate(
        [jnp.sum(s1, axis=0)[None], jnp.sum(s2, axis=0)[None],
         jnp.zeros((6, C), jnp.float32)], axis=0)


def _conv_call(gath, h, wct, wnt, bias, Dp, M=128):
    grid = (B, N // M)
    nt = N // M
    return pl.pallas_call(
        functools.partial(_conv_body, M=M, Dp=Dp),
        grid=grid,
        in_specs=[
            pl.BlockSpec((1, M * KNN, Dp), lambda b, t: (b, t, 0)),
            pl.BlockSpec((1, M, Dp), lambda b, t: (b, t, 0)),
            pl.BlockSpec((Dp, C), lambda b, t: (0, 0)),
            pl.BlockSpec((Dp, C), lambda b, t: (0, 0)),
            pl.BlockSpec((1, C), lambda b, t: (0, 0)),
        ],
        out_specs=[
            pl.BlockSpec((1, M, C), lambda b, t: (b, t, 0)),
            pl.BlockSpec((1, M, C), lambda b, t: (b, t, 0)),
            pl.BlockSpec((1, 1, 8, C), lambda b, t: (b, t, 0, 0)),
        ],
        out_shape=[
            jax.ShapeDtypeStruct((B, N, C), jnp.float32),
            jax.ShapeDtypeStruct((B, N, C), jnp.float32),
            jax.ShapeDtypeStruct((B, nt, 8, C), jnp.float32),
        ],
    )(gath, h, wct, wnt, bias)


def _fin_body(mx_ref, mn_ref, sc_ref, sh_ref, o_ref):
    sc = sc_ref[...][0][None, :]
    sh = sh_ref[...][0][None, :]
    sel = jnp.where(sc >= 0, mx_ref[0], mn_ref[0])
    o_ref[0] = jnp.maximum(sc * sel + sh, 0.0)


def _fin_call(mx, mn, scale, shift, M=512):
    ch = mx.shape[-1]
    grid = (B, N // M)
    return pl.pallas_call(
        _fin_body,
        grid=grid,
        in_specs=[
            pl.BlockSpec((1, M, ch), lambda b, t: (b, t, 0)),
            pl.BlockSpec((1, M, ch), lambda b, t: (b, t, 0)),
            pl.BlockSpec((1, ch), lambda b, t: (0, 0)),
            pl.BlockSpec((1, ch), lambda b, t: (0, 0)),
        ],
        out_specs=pl.BlockSpec((1, M, ch), lambda b, t: (b, t, 0)),
        out_shape=jax.ShapeDtypeStruct((B, N, ch), jnp.float32),
    )(mx, mn, scale[None], shift[None])


def _fconv_body(x_ref, wt_ref, bias_ref, y_ref, ps_ref, *, M, Cin, Cout):
    y = lax.dot_general(x_ref[0].astype(jnp.bfloat16), wt_ref[...],
                        (((1,), (0,)), ((), ())),
                        preferred_element_type=jnp.float32)
    y = y + bias_ref[...][0][None, :]
    y_ref[0] = y
    ps_ref[0, 0] = jnp.concatenate(
        [jnp.sum(y, axis=0)[None], jnp.sum(y * y, axis=0)[None],
         jnp.zeros((6, Cout), jnp.float32)], axis=0)


def _fconv_call(x, wt, bias, M=256):
    Cin, Cout = wt.shape
    grid = (B, N // M)
    nt = N // M
    return pl.pallas_call(
        functools.partial(_fconv_body, M=M, Cin=Cin, Cout=Cout),
        grid=grid,
        in_specs=[
            pl.BlockSpec((1, M, Cin), lambda b, t: (b, t, 0)),
            pl.BlockSpec((Cin, Cout), lambda b, t: (0, 0)),
            pl.BlockSpec((1, Cout), lambda b, t: (0, 0)),
        ],
        out_specs=[
            pl.BlockSpec((1, M, Cout), lambda b, t: (b, t, 0)),
            pl.BlockSpec((1, 1, 8, Cout), lambda b, t: (b, t, 0, 0)),
        ],
        out_shape=[
            jax.ShapeDtypeStruct((B, N, Cout), jnp.float32),
            jax.ShapeDtypeStruct((B, nt, 8, Cout), jnp.float32),
        ],
    )(x, wt, bias[None])


def _ffin_body(y_ref, sc_ref, sh_ref, o_ref):
    sc = sc_ref[...][0][None, :]
    sh = sh_ref[...][0][None, :]
    o_ref[0] = jnp.maximum(y_ref[0] * sc + sh, 0.0)


def _ffin_call(y, scale, shift, M=512):
    ch = y.shape[-1]
    grid = (B, N // M)
    return pl.pallas_call(
        _ffin_body,
        grid=grid,
        in_specs=[
            pl.BlockSpec((1, M, ch), lambda b, t: (b, t, 0)),
            pl.BlockSpec((1, ch), lambda b, t: (0, 0)),
            pl.BlockSpec((1, ch), lambda b, t: (0, 0)),
        ],
        out_specs=pl.BlockSpec((1, M, ch), lambda b, t: (b, t, 0)),
        out_shape=jax.ShapeDtypeStruct((B, N, ch), jnp.float32),
    )(y, scale[None], shift[None])


def _edge_layer(h, W, bvec, g, e, Din, Dp):
    Wc = W[:, :Din]
    Wn = W[:, Din:]
    pad = Dp - Din
    if pad:
        Wc = jnp.pad(Wc, ((0, 0), (0, pad)))
        Wn = jnp.pad(Wn, ((0, 0), (0, pad)))
    wct = Wc.T.astype(jnp.bfloat16)
    wnt = Wn.T.astype(jnp.bfloat16)

    idx = _topk_call(h, Dp)
    gath = _sc_gather(h.reshape(B * N, Dp), idx.reshape(-1), Dp)
    gath = gath.reshape(B, N * KNN, Dp)
    mx, mn, ps = _conv_call(gath, h, wct, wnt, bvec[None], Dp)

    cnt = jnp.float32(B * N * KNN)
    s1 = jnp.sum(ps[:, :, 0, :], axis=(0, 1))
    s2 = jnp.sum(ps[:, :, 1, :], axis=(0, 1))
    mean = s1 / cnt
    var = s2 / cnt - mean * mean
    scale = g / jnp.sqrt(var + EPS)
    shift = e - mean * scale
    return _fin_call(mx, mn, scale, shift)


def kernel(x, W0, b0, g0, e0, W1, b1, g1, e1, W2, b2, g2, e2, Wf, bf, gf, ef):
    h0 = jnp.transpose(x, (0, 2, 1))
    h0 = jnp.pad(h0, ((0, 0), (0, 0), (0, 13)))

    h1 = _edge_layer(h0, W0, b0, g0, e0, 3, 16)
    h2 = _edge_layer(h1, W1, b1, g1, e1, 64, 64)
    h3 = _edge_layer(h2, W2, b2, g2, e2, 64, 64)

    cat = jnp.concatenate([h1, h2, h3], axis=-1)
    wft = Wf.T.astype(jnp.bfloat16)
    y, ps = _fconv_call(cat, wft, bf)
    cnt = jnp.float32(B * N)
    mean = jnp.sum(ps[:, :, 0, :], axis=(0, 1)) / cnt
    var = jnp.sum(ps[:, :, 1, :], axis=(0, 1)) / cnt - mean * mean
    scale = gf / jnp.sqrt(var + EPS)
    shift = ef - mean * scale
    out = _ffin_call(y, scale, shift)
    return jnp.transpose(out, (0, 2, 1))


# speedup vs baseline: 7.3226x; 1.1282x over previous
import functools

import jax
import jax.numpy as jnp
from jax import lax
from jax.experimental import pallas as pl
from jax.experimental.pallas import tpu as pltpu
from jax.experimental.pallas import tpu_sc as plsc

KNN = 20
EPS = 1e-5
B, N = 16, 2048
C = 64

_NC, _NS = 2, 16
_NW = _NC * _NS


def _full_extract(dist, base, M):
    iota = lax.broadcasted_iota(jnp.int32, (M, N), 1)
    kiota = lax.broadcasted_iota(jnp.int32, (M, 32), 1)

    def step(t, carry):
        work, acc = carry
        m = jnp.min(work, axis=1, keepdims=True)
        cand = jnp.where(work == m, iota, N)
        j = jnp.min(cand, axis=1, keepdims=True)
        acc = jnp.where(kiota == t, j + base, acc)
        work = jnp.where(iota == j, jnp.float32(jnp.inf), work)
        return work, acc

    _, acc = lax.fori_loop(0, KNN, step, (dist, jnp.zeros((M, 32), jnp.int32)))
    return acc[:, :KNN]


def _topk_body(hr_ref, hf_ref, idx_ref, *, M, Dp):
    hr = hr_ref[0]
    hf = hf_ref[0]
    b = pl.program_id(0)
    sq_r = jnp.sum(hr * hr, axis=1, keepdims=True)
    sq_f = jnp.sum(hf * hf, axis=1)[None, :]
    s = lax.dot_general(hr.astype(jnp.bfloat16), hf.astype(jnp.bfloat16),
                        (((1,), (1,)), ((), ())),
                        preferred_element_type=jnp.float32)
    dist = (sq_r + (-2.0 * s)) + sq_f
    base = b * N

    G, L = 16, 128
    d3 = dist.reshape(M, G, L)
    tio = lax.broadcasted_iota(jnp.int32, (M, G, L), 1)
    lio = lax.broadcasted_iota(jnp.int32, (M, 1, L), 2)
    w3 = d3
    cv, ci = [], []
    for _ in range(4):
        gm = jnp.min(w3, axis=1, keepdims=True)
        tc = jnp.where(w3 == gm, tio, G)
        tm = jnp.min(tc, axis=1, keepdims=True)
        cv.append(gm)
        ci.append(tm * L + lio)
        w3 = jnp.where(tio == tm, jnp.float32(jnp.inf), w3)
    candv = jnp.concatenate(cv, axis=1).reshape(M, 4 * L)
    candi = jnp.concatenate(ci, axis=1).reshape(M, 4 * L)

    kiota = lax.broadcasted_iota(jnp.int32, (M, 32), 1)

    def step(t, carry):
        work, acc, _ = carry
        m = jnp.min(work, axis=1, keepdims=True)
        cand = jnp.where(work == m, candi, N)
        j = jnp.min(cand, axis=1, keepdims=True)
        acc = jnp.where(kiota == t, j + base, acc)
        work = jnp.where(candi == j, jnp.float32(jnp.inf), work)
        return work, acc, m

    _, acc, vlast = lax.fori_loop(
        0, KNN, step,
        (candv, jnp.zeros((M, 32), jnp.int32), jnp.zeros((M, 1), jnp.float32)))

    cnt = jnp.sum((d3 <= vlast[:, :, None]).astype(jnp.int32), axis=1)
    fail = jnp.any(cnt > 4)
    idx_ref[0] = lax.cond(fail,
                          lambda: _full_extract(dist, base, M),
                          lambda: acc[:, :KNN])


def _topk_call(h, Dp, M=128):
    grid = (B, N // M)
    return pl.pallas_call(
        functools.partial(_topk_body, M=M, Dp=Dp),
        grid=grid,
        in_specs=[
            pl.BlockSpec((1, M, Dp), lambda b, t: (b, t, 0)),
            pl.BlockSpec((1, N, Dp), lambda b, t: (b, 0, 0)),
        ],
        out_specs=pl.BlockSpec((1, M, KNN), lambda b, t: (b, t, 0)),
        out_shape=jax.ShapeDtypeStruct((B, N, KNN), jnp.int32),
    )(h, h)


def _sc_gather(table, idxf, Dp):
    P = 64
    CHUNK = P * KNN
    per_w = (B * N) // _NW
    n_chunks = per_w // P
    SUB = 128
    n_sub = CHUNK // SUB

    mesh = plsc.VectorSubcoreMesh(core_axis_name="c", subcore_axis_name="s")

    @functools.partial(
        pl.kernel,
        out_type=jax.ShapeDtypeStruct((B * N * KNN, Dp), jnp.float32),
        mesh=mesh,
        compiler_params=pltpu.CompilerParams(use_tc_tiling_on_sc=False),
        scratch_types=[
            pltpu.VMEM((CHUNK,), jnp.int32),
            pltpu.VMEM((CHUNK, Dp), jnp.float32),
            pltpu.SemaphoreType.DMA,
        ],
    )
    def gk(table_hbm, idx_hbm, out_hbm, idx_v, rows_v, sem):
        wid = lax.axis_index("s") * _NC + lax.axis_index("c")
        w_off = wid * per_w * KNN

        def chunk_body(ci, _):
            off = w_off + ci * CHUNK
            pltpu.sync_copy(idx_hbm.at[pl.ds(off, CHUNK)], idx_v)
            descs = []
            for j in range(n_sub):
                descs.append(pltpu.async_copy(
                    table_hbm.at[idx_v.at[pl.ds(j * SUB, SUB)]],
                    rows_v.at[pl.ds(j * SUB, SUB)],
                    sem))
            for d in descs:
                d.wait()
            pltpu.sync_copy(rows_v, out_hbm.at[pl.ds(off, CHUNK)])
            return 0

        lax.fori_loop(0, n_chunks, chunk_body, 0)

    return gk(table, idxf)


def _conv_body(g_ref, h_ref, wct_ref, wnt_ref, bias_ref, mx_ref, mn_ref, ps_ref,
               *, M, Dp):
    hr = h_ref[0]
    cen = lax.dot_general(hr.astype(jnp.bfloat16), wct_ref[...],
                          (((1,), (0,)), ((), ())),
                          preferred_element_type=jnp.float32)
    cen = cen + bias_ref[...][0][None, :]
    g3 = g_ref[0].reshape(M, KNN, Dp)
    diff = (g3 - hr[:, None, :]).astype(jnp.bfloat16).reshape(M * KNN, Dp)
    yn = lax.dot_general(diff, wnt_ref[...],
                         (((1,), (0,)), ((), ())),
                         preferred_element_type=jnp.float32)
    y = yn.reshape(M, KNN, C) + cen[:, None, :]
    s1 = jnp.sum(y, axis=1)
    s2 = jnp.sum(y * y, axis=1)
    mx_ref[0] = jnp.max(y, axis=1)
    mn_ref[0] = jnp.min(y, axis=1)
    ps_ref[0, 0] = jnp.concatenate(
        [jnp.sum(s1, axis=0)[None], jnp.sum(s2, axis=0)[None],
         jnp.zeros((6, C), jnp.float32)], axis=0)


def _conv_call(gath, h, wct, wnt, bias, Dp, M=128):
    grid = (B, N // M)
    nt = N // M
    return pl.pallas_call(
        functools.partial(_conv_body, M=M, Dp=Dp),
        grid=grid,
        in_specs=[
            pl.BlockSpec((1, M * KNN, Dp), lambda b, t: (b, t, 0)),
            pl.BlockSpec((1, M, Dp), lambda b, t: (b, t, 0)),
            pl.BlockSpec((Dp, C), lambda b, t: (0, 0)),
            pl.BlockSpec((Dp, C), lambda b, t: (0, 0)),
            pl.BlockSpec((1, C), lambda b, t: (0, 0)),
        ],
        out_specs=[
            pl.BlockSpec((1, M, C), lambda b, t: (b, t, 0)),
            pl.BlockSpec((1, M, C), lambda b, t: (b, t, 0)),
            pl.BlockSpec((1, 1, 8, C), lambda b, t: (b, t, 0, 0)),
        ],
        out_shape=[
            jax.ShapeDtypeStruct((B, N, C), jnp.float32),
            jax.ShapeDtypeStruct((B, N, C), jnp.float32),
            jax.ShapeDtypeStruct((B, nt, 8, C), jnp.float32),
        ],
    )(gath, h, wct, wnt, bias)


def _fin_body(mx_ref, mn_ref, sc_ref, sh_ref, o_ref):
    sc = sc_ref[...][0][None, :]
    sh = sh_ref[...][0][None, :]
    sel = jnp.where(sc >= 0, mx_ref[0], mn_ref[0])
    o_ref[0] = jnp.maximum(sc * sel + sh, 0.0)


def _fin_call(mx, mn, scale, shift, M=512):
    ch = mx.shape[-1]
    grid = (B, N // M)
    return pl.pallas_call(
        _fin_body,
        grid=grid,
        in_specs=[
            pl.BlockSpec((1, M, ch), lambda b, t: (b, t, 0)),
            pl.BlockSpec((1, M, ch), lambda b, t: (b, t, 0)),
            pl.BlockSpec((1, ch), lambda b, t: (0, 0)),
            pl.BlockSpec((1, ch), lambda b, t: (0, 0)),
        ],
        out_specs=pl.BlockSpec((1, M, ch), lambda b, t: (b, t, 0)),
        out_shape=jax.ShapeDtypeStruct((B, N, ch), jnp.float32),
    )(mx, mn, scale[None], shift[None])


def _fconv_body(x_ref, wt_ref, bias_ref, y_ref, ps_ref, *, M, Cin, Cout):
    y = lax.dot_general(x_ref[0].astype(jnp.bfloat16), wt_ref[...],
                        (((1,), (0,)), ((), ())),
                        preferred_element_type=jnp.float32)
    y = y + bias_ref[...][0][None, :]
    y_ref[0] = y
    ps_ref[0, 0] = jnp.concatenate(
        [jnp.sum(y, axis=0)[None], jnp.sum(y * y, axis=0)[None],
         jnp.zeros((6, Cout), jnp.float32)], axis=0)


def _fconv_call(x, wt, bias, M=256):
    Cin, Cout = wt.shape
    grid = (B, N // M)
    nt = N // M
    return pl.pallas_call(
        functools.partial(_fconv_body, M=M, Cin=Cin, Cout=Cout),
        grid=grid,
        in_specs=[
            pl.BlockSpec((1, M, Cin), lambda b, t: (b, t, 0)),
            pl.BlockSpec((Cin, Cout), lambda b, t: (0, 0)),
            pl.BlockSpec((1, Cout), lambda b, t: (0, 0)),
        ],
        out_specs=[
            pl.BlockSpec((1, M, Cout), lambda b, t: (b, t, 0)),
            pl.BlockSpec((1, 1, 8, Cout), lambda b, t: (b, t, 0, 0)),
        ],
        out_shape=[
            jax.ShapeDtypeStruct((B, N, Cout), jnp.float32),
            jax.ShapeDtypeStruct((B, nt, 8, Cout), jnp.float32),
        ],
    )(x, wt, bias[None])


def _ffin_body(y_ref, sc_ref, sh_ref, o_ref):
    sc = sc_ref[...][0][None, :]
    sh = sh_ref[...][0][None, :]
    o_ref[0] = jnp.maximum(y_ref[0] * sc + sh, 0.0)


def _ffin_call(y, scale, shift, M=512):
    ch = y.shape[-1]
    grid = (B, N // M)
    return pl.pallas_call(
        _ffin_body,
        grid=grid,
        in_specs=[
            pl.BlockSpec((1, M, ch), lambda b, t: (b, t, 0)),
            pl.BlockSpec((1, ch), lambda b, t: (0, 0)),
            pl.BlockSpec((1, ch), lambda b, t: (0, 0)),
        ],
        out_specs=pl.BlockSpec((1, M, ch), lambda b, t: (b, t, 0)),
        out_shape=jax.ShapeDtypeStruct((B, N, ch), jnp.float32),
    )(y, scale[None], shift[None])


def _edge_layer(h, W, bvec, g, e, Din, Dp):
    Wc = W[:, :Din]
    Wn = W[:, Din:]
    pad = Dp - Din
    if pad:
        Wc = jnp.pad(Wc, ((0, 0), (0, pad)))
        Wn = jnp.pad(Wn, ((0, 0), (0, pad)))
    wct = Wc.T.astype(jnp.bfloat16)
    wnt = Wn.T.astype(jnp.bfloat16)

    idx = _topk_call(h, Dp)
    gath = _sc_gather(h.reshape(B * N, Dp), idx.reshape(-1), Dp)
    gath = gath.reshape(B, N * KNN, Dp)
    mx, mn, ps = _conv_call(gath, h, wct, wnt, bvec[None], Dp)

    cnt = jnp.float32(B * N * KNN)
    s1 = jnp.sum(ps[:, :, 0, :], axis=(0, 1))
    s2 = jnp.sum(ps[:, :, 1, :], axis=(0, 1))
    mean = s1 / cnt
    var = s2 / cnt - mean * mean
    scale = g / jnp.sqrt(var + EPS)
    shift = e - mean * scale
    return _fin_call(mx, mn, scale, shift)


def kernel(x, W0, b0, g0, e0, W1, b1, g1, e1, W2, b2, g2, e2, Wf, bf, gf, ef):
    h0 = jnp.transpose(x, (0, 2, 1))
    h0 = jnp.pad(h0, ((0, 0), (0, 0), (0, 13)))

    h1 = _edge_layer(h0, W0, b0, g0, e0, 3, 16)
    h2 = _edge_layer(h1, W1, b1, g1, e1, 64, 64)
    h3 = _edge_layer(h2, W2, b2, g2, e2, 64, 64)

    cat = jnp.concatenate([h1, h2, h3], axis=-1)
    wft = Wf.T.astype(jnp.bfloat16)
    y, ps = _fconv_call(cat, wft, bf)
    cnt = jnp.float32(B * N)
    mean = jnp.sum(ps[:, :, 0, :], axis=(0, 1)) / cnt
    var = jnp.sum(ps[:, :, 1, :], axis=(0, 1)) / cnt - mean * mean
    scale = gf / jnp.sqrt(var + EPS)
    shift = ef - mean * scale
    out = _ffin_call(y, scale, shift)
    return jnp.transpose(out, (0, 2, 1))
